# baseline (speedup 1.0000x reference)
import functools

import jax
import jax.numpy as jnp
from jax import lax
from jax.experimental import pallas as pl
from jax.experimental.pallas import tpu as pltpu
from jax.experimental.pallas import tpu_sc as plsc

EMBED_DIM = 32
CHUNK = 1024
NBUF = 2
N_WORKERS = 32


def _gather_kernel(n_rows, n_total, n_chunks):
    mesh = plsc.VectorSubcoreMesh(core_axis_name="c", subcore_axis_name="s")
    b_per_w = n_chunks * CHUNK

    @functools.partial(
        pl.kernel,
        mesh=mesh,
        out_type=jax.ShapeDtypeStruct((n_total, EMBED_DIM), jnp.float32),
        scratch_types=[
            pltpu.VMEM((b_per_w,), jnp.int32),
            pltpu.VMEM((CHUNK, EMBED_DIM), jnp.float32),
            pltpu.VMEM((CHUNK, EMBED_DIM), jnp.float32),
            pltpu.SemaphoreType.DMA,
            pltpu.SemaphoreType.DMA,
            pltpu.SemaphoreType.DMA,
            pltpu.SemaphoreType.DMA,
        ],
        compiler_params=pltpu.CompilerParams(use_tc_tiling_on_sc=False),
    )
    def k(idx_hbm, table_hbm, out_hbm, idx_v, rows0, rows1, g0, g1, w0, w1):
        wid = lax.axis_index("s") * 2 + lax.axis_index("c")
        base = wid * b_per_w
        rows = (rows0, rows1)
        gsem = (g0, g1)
        wsem = (w0, w1)

        pltpu.sync_copy(idx_hbm.at[pl.ds(base, b_per_w)], idx_v)

        def start_gather(g, b):
            pltpu.async_copy(
                table_hbm.at[idx_v.at[pl.ds(g * CHUNK, CHUNK)]],
                rows[b],
                gsem[b],
            )

        def wait_gather(b):
            pltpu.make_async_copy(
                table_hbm.at[idx_v.at[pl.ds(0, CHUNK)]], rows[b], gsem[b]
            ).wait()

        def start_wb(g, b):
            pltpu.async_copy(
                rows[b], out_hbm.at[pl.ds(base + g * CHUNK, CHUNK)], wsem[b]
            )

        def wait_wb(b):
            pltpu.make_async_copy(
                rows[b], out_hbm.at[pl.ds(0, CHUNK)], wsem[b]
            ).wait()

        for b in range(NBUF):
            start_gather(b, b)

        def body(o, carry):
            for b in range(NBUF):
                g = o * NBUF + b
                wait_gather(b)
                start_wb(g, b)
                wait_wb(b)
                start_gather(g + NBUF, b)
            return carry

        lax.fori_loop(0, n_chunks // NBUF - 1, body, 0)

        for b in range(NBUF):
            wait_gather(b)
            start_wb(n_chunks - NBUF + b, b)
        for b in range(NBUF):
            wait_wb(b)

    return k


def kernel(indices, table):
    batch, hist = indices.shape
    n_total = batch * hist
    n_chunks = n_total // (N_WORKERS * CHUNK)

    flat_idx = indices.reshape(n_total).astype(jnp.int32)
    out = _gather_kernel(table.shape[0], n_total, n_chunks)(flat_idx, table)
    return out.reshape(batch, hist, EMBED_DIM)


# speedup vs baseline: 1.1017x; 1.1017x over previous
import functools

import jax
import jax.numpy as jnp
from jax import lax
from jax.experimental import pallas as pl
from jax.experimental.pallas import tpu as pltpu
from jax.experimental.pallas import tpu_sc as plsc

EMBED_DIM = 32
MAX_ID = 1000000
CHUNK = 1024
NBUF = 2
N_WORKERS = 32

E_BLK = 2048
N_BLK = (MAX_ID + E_BLK - 1) // E_BLK
N_VIRT = N_BLK * E_BLK


def _tc_transpose():
    def tk(x_ref, o_ref):
        t = x_ref[...].T
        o_ref[...] = jnp.concatenate(
            [t[512 * u : 512 * (u + 1), :] for u in range(4)], axis=1
        )

    return pl.pallas_call(
        tk,
        grid=(N_BLK,),
        in_specs=[pl.BlockSpec((EMBED_DIM, E_BLK), lambda i: (0, i))],
        out_specs=pl.BlockSpec((E_BLK // 4, 128), lambda i: (i, 0)),
        out_shape=jax.ShapeDtypeStruct(
            (N_VIRT * EMBED_DIM // 128, 128), jnp.float32
        ),
    )


def _gather_kernel(n_total, n_chunks):
    mesh = plsc.VectorSubcoreMesh(core_axis_name="c", subcore_axis_name="s")
    b_per_w = n_chunks * CHUNK

    @functools.partial(
        pl.kernel,
        mesh=mesh,
        out_type=jax.ShapeDtypeStruct((n_total, EMBED_DIM), jnp.float32),
        scratch_types=[
            pltpu.VMEM((b_per_w,), jnp.int32),
            pltpu.VMEM((CHUNK, EMBED_DIM), jnp.float32),
            pltpu.VMEM((CHUNK, EMBED_DIM), jnp.float32),
            pltpu.SemaphoreType.DMA,
            pltpu.SemaphoreType.DMA,
            pltpu.SemaphoreType.DMA,
            pltpu.SemaphoreType.DMA,
        ],
        compiler_params=pltpu.CompilerParams(use_tc_tiling_on_sc=False),
    )
    def k(idx_hbm, table_hbm, out_hbm, idx_v, rows0, rows1, g0, g1, w0, w1):
        wid = lax.axis_index("s") * 2 + lax.axis_index("c")
        base = wid * b_per_w
        rows = (rows0, rows1)
        gsem = (g0, g1)
        wsem = (w0, w1)

        pltpu.sync_copy(idx_hbm.at[pl.ds(base, b_per_w)], idx_v)

        def remap(k, carry):
            v = idx_v[pl.ds(k * 16, 16)]
            idx_v[pl.ds(k * 16, 16)] = (
                (v & ~jnp.int32(2047))
                | ((v & jnp.int32(511)) << 2)
                | ((v & jnp.int32(2047)) >> 9)
            )
            return carry

        lax.fori_loop(0, b_per_w // 16, remap, 0, unroll=8)

        def start_gather(g, b):
            pltpu.async_copy(
                table_hbm.at[idx_v.at[pl.ds(g * CHUNK, CHUNK)]],
                rows[b],
                gsem[b],
            )

        def wait_gather(b):
            pltpu.make_async_copy(
                table_hbm.at[idx_v.at[pl.ds(0, CHUNK)]], rows[b], gsem[b]
            ).wait()

        def start_wb(g, b):
            pltpu.async_copy(
                rows[b], out_hbm.at[pl.ds(base + g * CHUNK, CHUNK)], wsem[b]
            )

        def wait_wb(b):
            pltpu.make_async_copy(
                rows[b], out_hbm.at[pl.ds(0, CHUNK)], wsem[b]
            ).wait()

        for b in range(NBUF):
            start_gather(b, b)

        def body(o, carry):
            for b in range(NBUF):
                g = o * NBUF + b
                wait_gather(b)
                start_wb(g, b)
                wait_wb(b)
                start_gather(g + NBUF, b)
            return carry

        lax.fori_loop(0, n_chunks // NBUF - 1, body, 0)

        for b in range(NBUF):
            wait_gather(b)
            start_wb(n_chunks - NBUF + b, b)
        for b in range(NBUF):
            wait_wb(b)

    return k


def kernel(indices, table):
    batch, hist = indices.shape
    n_total = batch * hist
    n_chunks = n_total // (N_WORKERS * CHUNK)

    tbl_t = jnp.swapaxes(table, 0, 1)
    tbl_rm = _tc_transpose()(tbl_t).reshape(N_VIRT, EMBED_DIM)

    flat_idx = indices.reshape(n_total).astype(jnp.int32)
    out = _gather_kernel(n_total, n_chunks)(flat_idx, tbl_rm)
    return out.reshape(batch, hist, EMBED_DIM)


# speedup vs baseline: 1.4869x; 1.3496x over previous
import functools

import jax
import jax.numpy as jnp
from jax import lax
from jax.experimental import pallas as pl
from jax.experimental.pallas import tpu as pltpu
from jax.experimental.pallas import tpu_sc as plsc

EMBED_DIM = 32
MAX_ID = 1000000
CHUNK = 1024
NBUF = 2
N_WORKERS = 32

E_BLK = 4096
N_BLK = (MAX_ID + E_BLK - 1) // E_BLK
N_VIRT = N_BLK * E_BLK
SLICE = E_BLK // 4
SH = SLICE.bit_length() - 1


def _tc_transpose():
    def tk(x_ref, o_ref):
        t = x_ref[...].T
        o_ref[...] = jnp.concatenate(
            [t[SLICE * u : SLICE * (u + 1), :] for u in range(4)], axis=1
        )

    return pl.pallas_call(
        tk,
        grid=(N_BLK,),
        in_specs=[pl.BlockSpec((EMBED_DIM, E_BLK), lambda i: (0, i))],
        out_specs=pl.BlockSpec((E_BLK // 4, 128), lambda i: (i, 0)),
        out_shape=jax.ShapeDtypeStruct(
            (N_VIRT * EMBED_DIM // 128, 128), jnp.float32
        ),
    )


def _gather_kernel(n_total, n_chunks):
    mesh = plsc.VectorSubcoreMesh(core_axis_name="c", subcore_axis_name="s")
    b_per_w = n_chunks * CHUNK

    @functools.partial(
        pl.kernel,
        mesh=mesh,
        out_type=jax.ShapeDtypeStruct((n_total, EMBED_DIM), jnp.float32),
        scratch_types=[
            pltpu.VMEM((b_per_w,), jnp.int32),
            pltpu.VMEM((CHUNK, EMBED_DIM), jnp.float32),
            pltpu.VMEM((CHUNK, EMBED_DIM), jnp.float32),
            pltpu.SemaphoreType.DMA,
            pltpu.SemaphoreType.DMA,
            pltpu.SemaphoreType.DMA,
            pltpu.SemaphoreType.DMA,
        ],
        compiler_params=pltpu.CompilerParams(use_tc_tiling_on_sc=False),
    )
    def k(idx_hbm, table_hbm, out_hbm, idx_v, rows0, rows1, g0, g1, w0, w1):
        wid = lax.axis_index("s") * 2 + lax.axis_index("c")
        base = wid * b_per_w
        rows = (rows0, rows1)
        gsem = (g0, g1)
        wsem = (w0, w1)

        pltpu.sync_copy(idx_hbm.at[pl.ds(base, b_per_w)], idx_v)

        def remap(k, carry):
            v = idx_v[pl.ds(k * 16, 16)]
            idx_v[pl.ds(k * 16, 16)] = (
                (v & ~jnp.int32(E_BLK - 1))
                | ((v & jnp.int32(SLICE - 1)) << 2)
                | ((v & jnp.int32(E_BLK - 1)) >> SH)
            )
            return carry

        lax.fori_loop(0, b_per_w // 16, remap, 0, unroll=8)

        def start_gather(g, b):
            pltpu.async_copy(
                table_hbm.at[idx_v.at[pl.ds(g * CHUNK, CHUNK)]],
                rows[b],
                gsem[b],
            )

        def wait_gather(b):
            pltpu.make_async_copy(
                table_hbm.at[idx_v.at[pl.ds(0, CHUNK)]], rows[b], gsem[b]
            ).wait()

        def start_wb(g, b):
            pltpu.async_copy(
                rows[b], out_hbm.at[pl.ds(base + g * CHUNK, CHUNK)], wsem[b]
            )

        def wait_wb(b):
            pltpu.make_async_copy(
                rows[b], out_hbm.at[pl.ds(0, CHUNK)], wsem[b]
            ).wait()

        for b in range(NBUF):
            start_gather(b, b)

        def body(o, carry):
            for b in range(NBUF):
                g = o * NBUF + b
                wait_gather(b)
                start_wb(g, b)
                wait_wb(b)
                start_gather(g + NBUF, b)
            return carry

        lax.fori_loop(0, n_chunks // NBUF - 1, body, 0)

        for b in range(NBUF):
            wait_gather(b)
            start_wb(n_chunks - NBUF + b, b)
        for b in range(NBUF):
            wait_wb(b)

    return k


def kernel(indices, table):
    batch, hist = indices.shape
    n_total = batch * hist
    n_chunks = n_total // (N_WORKERS * CHUNK)

    tbl_t = jnp.swapaxes(table, 0, 1)
    tbl_rm = _tc_transpose()(tbl_t).reshape(N_VIRT, EMBED_DIM)

    flat_idx = jnp.swapaxes(indices, 0, 1).reshape(n_total).astype(jnp.int32)
    out = _gather_kernel(n_total, n_chunks)(flat_idx, tbl_rm)
    return jnp.swapaxes(out.reshape(hist, batch, EMBED_DIM), 0, 1)


# speedup vs baseline: 1.6324x; 1.0978x over previous
import functools

import jax
import jax.numpy as jnp
from jax import lax
from jax.experimental import pallas as pl
from jax.experimental.pallas import tpu as pltpu
from jax.experimental.pallas import tpu_sc as plsc

EMBED_DIM = 32
MAX_ID = 1000000
CHUNK = 1024
NBUF = 2
N_WORKERS = 32

E_BLK = 8192
N_BLK = (MAX_ID + E_BLK - 1) // E_BLK
N_VIRT = N_BLK * E_BLK
SLICE = E_BLK // 4
SH = SLICE.bit_length() - 1


def _tc_transpose():
    def tk(x_ref, o_ref):
        t = x_ref[...].T
        o_ref[...] = jnp.concatenate(
            [t[SLICE * u : SLICE * (u + 1), :] for u in range(4)], axis=1
        )

    return pl.pallas_call(
        tk,
        grid=(N_BLK,),
        in_specs=[pl.BlockSpec((EMBED_DIM, E_BLK), lambda i: (0, i))],
        out_specs=pl.BlockSpec((E_BLK // 4, 128), lambda i: (i, 0)),
        out_shape=jax.ShapeDtypeStruct(
            (N_VIRT * EMBED_DIM // 128, 128), jnp.float32
        ),
    )


def _gather_kernel(n_total, n_chunks):
    mesh = plsc.VectorSubcoreMesh(core_axis_name="c", subcore_axis_name="s")
    b_per_w = n_chunks * CHUNK

    @functools.partial(
        pl.kernel,
        mesh=mesh,
        out_type=jax.ShapeDtypeStruct((n_total, EMBED_DIM), jnp.float32),
        scratch_types=[
            pltpu.VMEM((b_per_w,), jnp.int32),
            pltpu.VMEM((CHUNK, EMBED_DIM), jnp.float32),
            pltpu.VMEM((CHUNK, EMBED_DIM), jnp.float32),
            pltpu.SemaphoreType.DMA,
            pltpu.SemaphoreType.DMA,
            pltpu.SemaphoreType.DMA,
            pltpu.SemaphoreType.DMA,
        ],
        compiler_params=pltpu.CompilerParams(use_tc_tiling_on_sc=False),
    )
    def k(idx_hbm, table_hbm, out_hbm, idx_v, rows0, rows1, g0, g1, w0, w1):
        wid = lax.axis_index("s") * 2 + lax.axis_index("c")
        base = wid * b_per_w
        rows = (rows0, rows1)
        gsem = (g0, g1)
        wsem = (w0, w1)

        pltpu.sync_copy(idx_hbm.at[pl.ds(base, b_per_w)], idx_v)

        def remap(k, carry):
            v = idx_v[pl.ds(k * 16, 16)]
            idx_v[pl.ds(k * 16, 16)] = (
                (v & ~jnp.int32(E_BLK - 1))
                | ((v & jnp.int32(SLICE - 1)) << 2)
                | ((v & jnp.int32(E_BLK - 1)) >> SH)
            )
            return carry

        lax.fori_loop(0, b_per_w // 16, remap, 0, unroll=8)

        def start_gather(g, b):
            pltpu.async_copy(
                table_hbm.at[idx_v.at[pl.ds(g * CHUNK, CHUNK)]],
                rows[b],
                gsem[b],
            )

        def wait_gather(b):
            pltpu.make_async_copy(
                table_hbm.at[idx_v.at[pl.ds(0, CHUNK)]], rows[b], gsem[b]
            ).wait()

        def start_wb(g, b):
            pltpu.async_copy(
                rows[b], out_hbm.at[pl.ds(base + g * CHUNK, CHUNK)], wsem[b]
            )

        def wait_wb(b):
            pltpu.make_async_copy(
                rows[b], out_hbm.at[pl.ds(0, CHUNK)], wsem[b]
            ).wait()

        for b in range(NBUF):
            start_gather(b, b)

        def body(o, carry):
            for b in range(NBUF):
                g = o * NBUF + b
                wait_gather(b)
                start_wb(g, b)
                wait_wb(b)
                start_gather(g + NBUF, b)
            return carry

        lax.fori_loop(0, n_chunks // NBUF - 1, body, 0)

        for b in range(NBUF):
            wait_gather(b)
            start_wb(n_chunks - NBUF + b, b)
        for b in range(NBUF):
            wait_wb(b)

    return k


def kernel(indices, table):
    batch, hist = indices.shape
    n_total = batch * hist
    n_chunks = n_total // (N_WORKERS * CHUNK)

    tbl_t = jnp.swapaxes(table, 0, 1)
    tbl_rm = _tc_transpose()(tbl_t).reshape(N_VIRT, EMBED_DIM)

    flat_idx = jnp.swapaxes(indices, 0, 1).reshape(n_total).astype(jnp.int32)
    out = _gather_kernel(n_total, n_chunks)(flat_idx, tbl_rm)
    return jnp.swapaxes(out.reshape(hist, batch, EMBED_DIM), 0, 1)


# speedup vs baseline: 1.6494x; 1.0104x over previous
import functools

import jax
import jax.numpy as jnp
from jax import lax
from jax.experimental import pallas as pl
from jax.experimental.pallas import tpu as pltpu
from jax.experimental.pallas import tpu_sc as plsc

EMBED_DIM = 32
MAX_ID = 1000000
CHUNK = 1024
NBUF = 2
N_WORKERS = 32

E_BLK = 16384
N_BLK = (MAX_ID + E_BLK - 1) // E_BLK
N_VIRT = N_BLK * E_BLK
SLICE = E_BLK // 4
SH = SLICE.bit_length() - 1


def _tc_transpose():
    def tk(x_ref, o_ref):
        t = x_ref[...].T
        o_ref[...] = jnp.concatenate(
            [t[SLICE * u : SLICE * (u + 1), :] for u in range(4)], axis=1
        )

    return pl.pallas_call(
        tk,
        grid=(N_BLK,),
        in_specs=[pl.BlockSpec((EMBED_DIM, E_BLK), lambda i: (0, i))],
        out_specs=pl.BlockSpec((E_BLK // 4, 128), lambda i: (i, 0)),
        out_shape=jax.ShapeDtypeStruct(
            (N_VIRT * EMBED_DIM // 128, 128), jnp.float32
        ),
    )


def _gather_kernel(n_total, n_chunks):
    mesh = plsc.VectorSubcoreMesh(core_axis_name="c", subcore_axis_name="s")
    b_per_w = n_chunks * CHUNK

    @functools.partial(
        pl.kernel,
        mesh=mesh,
        out_type=jax.ShapeDtypeStruct((n_total, EMBED_DIM), jnp.float32),
        scratch_types=[
            pltpu.VMEM((b_per_w,), jnp.int32),
            pltpu.VMEM((CHUNK, EMBED_DIM), jnp.float32),
            pltpu.VMEM((CHUNK, EMBED_DIM), jnp.float32),
            pltpu.SemaphoreType.DMA,
            pltpu.SemaphoreType.DMA,
            pltpu.SemaphoreType.DMA,
            pltpu.SemaphoreType.DMA,
        ],
        compiler_params=pltpu.CompilerParams(use_tc_tiling_on_sc=False),
    )
    def k(idx_hbm, table_hbm, out_hbm, idx_v, rows0, rows1, g0, g1, w0, w1):
        wid = lax.axis_index("s") * 2 + lax.axis_index("c")
        base = wid * b_per_w
        rows = (rows0, rows1)
        gsem = (g0, g1)
        wsem = (w0, w1)

        pltpu.sync_copy(idx_hbm.at[pl.ds(base, b_per_w)], idx_v)

        def remap(k, carry):
            v = idx_v[pl.ds(k * 16, 16)]
            idx_v[pl.ds(k * 16, 16)] = (
                (v & ~jnp.int32(E_BLK - 1))
                | ((v & jnp.int32(SLICE - 1)) << 2)
                | ((v & jnp.int32(E_BLK - 1)) >> SH)
            )
            return carry

        lax.fori_loop(0, b_per_w // 16, remap, 0, unroll=8)

        def start_gather(g, b):
            pltpu.async_copy(
                table_hbm.at[idx_v.at[pl.ds(g * CHUNK, CHUNK)]],
                rows[b],
                gsem[b],
            )

        def wait_gather(b):
            pltpu.make_async_copy(
                table_hbm.at[idx_v.at[pl.ds(0, CHUNK)]], rows[b], gsem[b]
            ).wait()

        def start_wb(g, b):
            pltpu.async_copy(
                rows[b], out_hbm.at[pl.ds(base + g * CHUNK, CHUNK)], wsem[b]
            )

        def wait_wb(b):
            pltpu.make_async_copy(
                rows[b], out_hbm.at[pl.ds(0, CHUNK)], wsem[b]
            ).wait()

        for b in range(NBUF):
            start_gather(b, b)

        def body(o, carry):
            for b in range(NBUF):
                g = o * NBUF + b
                wait_gather(b)
                start_wb(g, b)
                wait_wb(b)
                start_gather(g + NBUF, b)
            return carry

        lax.fori_loop(0, n_chunks // NBUF - 1, body, 0)

        for b in range(NBUF):
            wait_gather(b)
            start_wb(n_chunks - NBUF + b, b)
        for b in range(NBUF):
            wait_wb(b)

    return k


def kernel(indices, table):
    batch, hist = indices.shape
    n_total = batch * hist
    n_chunks = n_total // (N_WORKERS * CHUNK)

    tbl_t = jnp.swapaxes(table, 0, 1)
    tbl_rm = _tc_transpose()(tbl_t).reshape(N_VIRT, EMBED_DIM)

    flat_idx = jnp.swapaxes(indices, 0, 1).reshape(n_total).astype(jnp.int32)
    out = _gather_kernel(n_total, n_chunks)(flat_idx, tbl_rm)
    return jnp.swapaxes(out.reshape(hist, batch, EMBED_DIM), 0, 1)
